# Initial kernel scaffold; baseline (speedup 1.0000x reference)
import jax
import jax.numpy as jnp
from jax.experimental import pallas as pl


def kernel(inpt, weights):
    raise NotImplementedError("write your pallas kernel here")


# speedup vs baseline: 7.8245x; 7.8245x over previous
import functools

import jax
import jax.numpy as jnp
from jax import lax
from jax.experimental import pallas as pl
from jax.experimental.pallas import tpu as pltpu
from jax.experimental.pallas import tpu_sc as plsc

_VOCAB = 100000
_PAD = _VOCAB
_D = 64
_SW = 20
_L = 16


@functools.cache
def _make_kernel(num_words):
    info = plsc.get_sparse_core_info()
    nc, ns = info.num_cores, info.num_subcores
    nw = nc * ns
    wpw = num_words // nw
    cw = 4
    chunks = wpw // cw
    rpc = cw * _SW
    nd = _D // _L

    mesh = plsc.VectorSubcoreMesh(core_axis_name="c", subcore_axis_name="s")

    @functools.partial(
        pl.kernel,
        mesh=mesh,
        out_type=jax.ShapeDtypeStruct((nw, wpw, _D), jnp.float32),
        compiler_params=pltpu.CompilerParams(use_tc_tiling_on_sc=False),
        scratch_types=[
            pltpu.VMEM((wpw * _SW,), jnp.int32),
            pltpu.VMEM((_SW, wpw), jnp.int32),
            pltpu.VMEM((wpw + _L,), jnp.float32),
            pltpu.VMEM((rpc, _D), jnp.float32),
            pltpu.VMEM((rpc, _D), jnp.float32),
            pltpu.VMEM((wpw, _D), jnp.float32),
            pltpu.SemaphoreType.DMA,
            pltpu.SemaphoreType.DMA,
        ],
    )
    def k(idx_hbm, idxt_hbm, table_hbm, out_hbm, idx_v, idxt_v, scale_v,
          rows_a, rows_b, out_v, sem_a, sem_b):
        wid = lax.axis_index("s") * nc + lax.axis_index("c")
        pltpu.sync_copy(idx_hbm.at[wid], idx_v)
        pltpu.sync_copy(idxt_hbm.at[wid], idxt_v)

        def scale_body(g, carry):
            cnt = jnp.zeros((_L,), jnp.int32)
            for j in range(_SW):
                v = idxt_v[j, pl.ds(g * _L, _L)]
                cnt = cnt + jnp.where(v != _PAD, 1, 0)
            fl = jnp.maximum(cnt, 1).astype(jnp.float32)
            scale_v[pl.ds(g * _L, _L)] = 1.0 / fl
            return carry

        lax.fori_loop(0, wpw // _L, scale_body, 0)

        def gather_chunk(c, buf, sem):
            idx_slice = idx_v.at[pl.ds(c * rpc, rpc)]
            pltpu.async_copy(table_hbm.at[idx_slice], buf, sem)

        def wait_chunk(buf, sem):
            pltpu.make_async_copy(
                table_hbm.at[idx_v.at[pl.ds(0, rpc)]], buf, sem).wait()

        def process(c, buf, sv, wbase):
            for w in range(cw):
                word = c * cw + w
                acc = [jnp.zeros((_L,), jnp.float32) for _ in range(nd)]
                for r in range(_SW):
                    row = w * _SW + r
                    for d in range(nd):
                        acc[d] = acc[d] + buf[row, pl.ds(d * _L, _L)]
                s = sv[wbase + w]
                for d in range(nd):
                    out_v[word, pl.ds(d * _L, _L)] = acc[d] * s

        gather_chunk(0, rows_a, sem_a)
        gather_chunk(1, rows_b, sem_b)

        def pair_body(p, carry):
            c0 = 2 * p
            sv = scale_v[pl.ds(p * 2 * cw, _L)]
            wait_chunk(rows_a, sem_a)
            process(c0, rows_a, sv, 0)

            @pl.when(p < chunks // 2 - 1)
            def _():
                gather_chunk(c0 + 2, rows_a, sem_a)

            wait_chunk(rows_b, sem_b)
            process(c0 + 1, rows_b, sv, cw)

            @pl.when(p < chunks // 2 - 1)
            def _():
                gather_chunk(c0 + 3, rows_b, sem_b)

            return carry

        lax.fori_loop(0, chunks // 2, pair_body, 0)

        pltpu.sync_copy(out_v, out_hbm.at[wid])

    return k


def kernel(inpt, weights):
    b, s, w = inpt.shape
    num_words = b * s
    wpw = num_words // 32
    flat = inpt.astype(jnp.int32).reshape(num_words, w)
    idx = flat.reshape(32, wpw * w)
    idxt = flat.T.reshape(w, 32, wpw).transpose(1, 0, 2)
    out = _make_kernel(num_words)(idx, idxt, weights)
    return out.reshape(b, s, _D)


# speedup vs baseline: 8.8395x; 1.1297x over previous
import functools

import jax
import jax.numpy as jnp
from jax import lax
from jax.experimental import pallas as pl
from jax.experimental.pallas import tpu as pltpu
from jax.experimental.pallas import tpu_sc as plsc

_VOCAB = 100000
_PAD = _VOCAB
_D = 64
_SW = 20
_L = 16


@functools.cache
def _make_kernel(num_words):
    info = plsc.get_sparse_core_info()
    nc, ns = info.num_cores, info.num_subcores
    nw = nc * ns
    wpw = num_words // nw
    cw = 4
    chunks = wpw // cw
    rpc = cw * _SW
    nd = _D // _L

    mesh = plsc.VectorSubcoreMesh(core_axis_name="c", subcore_axis_name="s")

    @functools.partial(
        pl.kernel,
        mesh=mesh,
        out_type=jax.ShapeDtypeStruct((nw, wpw, _D), jnp.float32),
        compiler_params=pltpu.CompilerParams(use_tc_tiling_on_sc=False),
        scratch_types=[
            pltpu.VMEM((wpw * _SW,), jnp.int32),
            pltpu.VMEM((rpc, _D), jnp.float32),
            pltpu.VMEM((rpc, _D), jnp.float32),
            pltpu.VMEM((wpw, _D), jnp.float32),
            pltpu.SemaphoreType.DMA,
            pltpu.SemaphoreType.DMA,
        ],
    )
    def k(idx_hbm, table_hbm, out_hbm, idx_v, rows_a, rows_b, out_v,
          sem_a, sem_b):
        wid = lax.axis_index("s") * nc + lax.axis_index("c")
        pltpu.sync_copy(idx_hbm.at[wid], idx_v)

        lanes = lax.iota(jnp.int32, _L)
        perms = [jnp.bitwise_xor(lanes, sh) for sh in (1, 2, 4, 8)]

        def gather_chunk(c, buf, sem):
            idx_slice = idx_v.at[pl.ds(c * rpc, rpc)]
            pltpu.async_copy(table_hbm.at[idx_slice], buf, sem)

        def wait_chunk(buf, sem):
            pltpu.make_async_copy(
                table_hbm.at[idx_v.at[pl.ds(0, rpc)]], buf, sem).wait()

        def process(c, buf):
            marks = []
            for kk in range(rpc // _L):
                v = idx_v[pl.ds(c * rpc + kk * _L, _L)]
                marks.append(jnp.where(v != _PAD, 1, 0))
            for w in range(cw):
                word = c * cw + w
                lo, hi = w * _SW, (w + 1) * _SW
                cnt = None
                for kk in range(rpc // _L):
                    k0 = kk * _L
                    if k0 + _L <= lo or k0 >= hi:
                        continue
                    m = marks[kk]
                    if k0 < lo:
                        m = jnp.where(lanes >= (lo - k0), m, 0)
                    if k0 + _L > hi:
                        m = jnp.where(lanes < (hi - k0), m, 0)
                    cnt = m if cnt is None else cnt + m
                for perm in perms:
                    cnt = cnt + cnt.at[perm].get(mode="promise_in_bounds")
                s = 1.0 / jnp.maximum(cnt, 1).astype(jnp.float32)
                acc = [jnp.zeros((_L,), jnp.float32) for _ in range(nd)]
                for r in range(_SW):
                    row = w * _SW + r
                    for d in range(nd):
                        acc[d] = acc[d] + buf[row, pl.ds(d * _L, _L)]
                for d in range(nd):
                    out_v[word, pl.ds(d * _L, _L)] = acc[d] * s

        gather_chunk(0, rows_a, sem_a)
        gather_chunk(1, rows_b, sem_b)

        def pair_body(p, carry):
            c0 = 2 * p
            wait_chunk(rows_a, sem_a)
            process(c0, rows_a)

            @pl.when(p < chunks // 2 - 1)
            def _():
                gather_chunk(c0 + 2, rows_a, sem_a)

            wait_chunk(rows_b, sem_b)
            process(c0 + 1, rows_b)

            @pl.when(p < chunks // 2 - 1)
            def _():
                gather_chunk(c0 + 3, rows_b, sem_b)

            return carry

        lax.fori_loop(0, chunks // 2, pair_body, 0)

        pltpu.sync_copy(out_v, out_hbm.at[wid])

    return k


def kernel(inpt, weights):
    b, s, w = inpt.shape
    num_words = b * s
    wpw = num_words // 32
    idx = inpt.astype(jnp.int32).reshape(32, wpw * w)
    out = _make_kernel(num_words)(idx, weights)
    return out.reshape(b, s, _D)


# speedup vs baseline: 11.5164x; 1.3028x over previous
import functools

import jax
import jax.numpy as jnp
from jax import lax
from jax.experimental import pallas as pl
from jax.experimental.pallas import tpu as pltpu
from jax.experimental.pallas import tpu_sc as plsc

_VOCAB = 100000
_PAD = _VOCAB
_D = 64
_L = 16


@functools.cache
def _make_kernel(b, s, w):
    info = plsc.get_sparse_core_info()
    nc, ns = info.num_cores, info.num_subcores
    nw = nc * ns
    bpw = b // nw
    nd = _D // _L

    mesh = plsc.VectorSubcoreMesh(core_axis_name="c", subcore_axis_name="s")

    @functools.partial(
        pl.kernel,
        mesh=mesh,
        out_type=jax.ShapeDtypeStruct((nw, s, bpw, _D), jnp.float32),
        scratch_types=[
            pltpu.VMEM((s, w, bpw), jnp.int32),
            pltpu.VMEM((s, bpw, _D), jnp.float32),
            pltpu.SemaphoreType.DMA,
        ],
        compiler_params=pltpu.CompilerParams(use_tc_tiling_on_sc=False),
    )
    def k(idx_hbm, table_hbm, out_hbm, idx_v, acc_v, sem):
        wid = lax.axis_index("s") * nc + lax.axis_index("c")
        b0 = wid * bpw
        pltpu.sync_copy(idx_hbm.at[:, :, pl.ds(b0, bpw)], idx_v)

        zero = jnp.zeros((_L,), jnp.float32)

        def zero_body(si, carry):
            for bl in range(bpw):
                for d in range(nd):
                    acc_v[si, bl, pl.ds(d * _L, _L)] = zero
            return carry

        lax.fori_loop(0, s, zero_body, 0)

        def fire_body(si, carry):
            for wj in range(w):
                pltpu.async_copy(
                    table_hbm.at[idx_v.at[si, wj]], acc_v.at[si], sem,
                    add=True)
            return carry

        lax.fori_loop(0, s, fire_body, 0)

        def drain_body(si, carry):
            for wj in range(w):
                pltpu.make_async_copy(
                    table_hbm.at[idx_v.at[0, 0]], acc_v.at[0], sem).wait()
            return carry

        lax.fori_loop(0, s, drain_body, 0)

        def scale_body(si, carry):
            invs = []
            for bh in range(bpw // _L):
                cnt = jnp.zeros((_L,), jnp.int32)
                for wj in range(w):
                    v = idx_v[si, wj, pl.ds(bh * _L, _L)]
                    cnt = cnt + jnp.where(v != _PAD, 1, 0)
                invs.append(1.0 / jnp.maximum(cnt, 1).astype(jnp.float32))
            for bl in range(bpw):
                sc = invs[bl // _L][bl % _L]
                for d in range(nd):
                    out_hbm_row = acc_v[si, bl, pl.ds(d * _L, _L)] * sc
                    acc_v[si, bl, pl.ds(d * _L, _L)] = out_hbm_row
            return carry

        lax.fori_loop(0, s, scale_body, 0)

        pltpu.sync_copy(acc_v, out_hbm.at[wid])

    return k


def kernel(inpt, weights):
    b, s, w = inpt.shape
    nw = 32
    bpw = b // nw
    idx = jnp.transpose(inpt.astype(jnp.int32), (1, 2, 0))
    out = _make_kernel(b, s, w)(idx, weights)
    out = jnp.transpose(out, (0, 2, 1, 3)).reshape(b, s, _D)
    return out


# speedup vs baseline: 12.2659x; 1.0651x over previous
import functools

import jax
import jax.numpy as jnp
from jax import lax
from jax.experimental import pallas as pl
from jax.experimental.pallas import tpu as pltpu
from jax.experimental.pallas import tpu_sc as plsc

_VOCAB = 100000
_PAD = _VOCAB
_D = 64
_L = 16


@functools.cache
def _make_kernel(b, s, w):
    info = plsc.get_sparse_core_info()
    nc, ns = info.num_cores, info.num_subcores
    nw = nc * ns
    bpw = b // nw
    nd = _D // _L

    mesh = plsc.VectorSubcoreMesh(core_axis_name="c", subcore_axis_name="s")

    @functools.partial(
        pl.kernel,
        mesh=mesh,
        out_type=jax.ShapeDtypeStruct((nw, bpw, s, _D), jnp.float32),
        scratch_types=[
            pltpu.VMEM((s, w, bpw), jnp.int32),
            pltpu.VMEM((s, bpw, _D), jnp.float32),
            pltpu.SemaphoreType.DMA,
            pltpu.SemaphoreType.DMA,
        ],
        compiler_params=pltpu.CompilerParams(use_tc_tiling_on_sc=False),
    )
    def k(idx_hbm, table_hbm, out_hbm, idx_v, acc_v, sem, sem_out):
        wid = lax.axis_index("s") * nc + lax.axis_index("c")
        b0 = wid * bpw
        pltpu.sync_copy(idx_hbm.at[:, :, pl.ds(b0, bpw)], idx_v)

        zero = jnp.zeros((_L,), jnp.float32)

        def zero_body(si, carry):
            for bl in range(bpw):
                for d in range(nd):
                    acc_v[si, bl, pl.ds(d * _L, _L)] = zero
            return carry

        lax.fori_loop(0, s, zero_body, 0)

        def fire_body(si, carry):
            for wj in range(w):
                pltpu.async_copy(
                    table_hbm.at[idx_v.at[si, wj]], acc_v.at[si], sem,
                    add=True)
            return carry

        lax.fori_loop(0, s, fire_body, 0)

        def drain_body(si, carry):
            for wj in range(w):
                pltpu.make_async_copy(
                    table_hbm.at[idx_v.at[0, 0]], acc_v.at[0], sem).wait()
            return carry

        lax.fori_loop(0, s, drain_body, 0)

        def scale_body(si, carry):
            invs = []
            for bh in range(bpw // _L):
                cnt = jnp.zeros((_L,), jnp.int32)
                for wj in range(w):
                    v = idx_v[si, wj, pl.ds(bh * _L, _L)]
                    cnt = cnt + jnp.where(v != _PAD, 1, 0)
                invs.append(1.0 / jnp.maximum(cnt, 1).astype(jnp.float32))
            for bl in range(bpw):
                sc = invs[bl // _L][bl % _L]
                for d in range(nd):
                    scaled = acc_v[si, bl, pl.ds(d * _L, _L)] * sc
                    acc_v[si, bl, pl.ds(d * _L, _L)] = scaled
            pltpu.async_copy(acc_v.at[si], out_hbm.at[wid, :, si], sem_out)
            return carry

        lax.fori_loop(0, s, scale_body, 0)

        def out_drain_body(si, carry):
            pltpu.make_async_copy(
                acc_v.at[0], out_hbm.at[wid, :, 0], sem_out).wait()
            return carry

        lax.fori_loop(0, s, out_drain_body, 0)

    return k


def kernel(inpt, weights):
    b, s, w = inpt.shape
    nw = 32
    bpw = b // nw
    idx = jnp.transpose(inpt.astype(jnp.int32), (1, 2, 0))
    out = _make_kernel(b, s, w)(idx, weights)
    return out.reshape(b, s, _D)
